# Initial kernel scaffold; baseline (speedup 1.0000x reference)
import jax
import jax.numpy as jnp
from jax.experimental import pallas as pl


def kernel(x, edge_index, edge_weights, W1, b1, W2, b2, W3, b3):
    raise NotImplementedError("write your pallas kernel here")


# speedup vs baseline: 12.5598x; 12.5598x over previous
import jax
import jax.numpy as jnp
from jax import lax
from jax.experimental import pallas as pl
from jax.experimental.pallas import tpu as pltpu
from jax.experimental.pallas import tpu_sc as plsc

N = 10000
E = 320000
D = 128
DH = D // 2
NC, NS = 2, 16
NW = NC * NS
C = 80
EPT32 = E // NW
NCH32 = EPT32 // C
EPT16 = E // NS
NCH16 = EPT16 // C
NROW = 640
N_PAD = NS * NROW
RB = 1000

_MESH = plsc.VectorSubcoreMesh(core_axis_name="c", subcore_axis_name="s")
_SC_PARAMS = pltpu.CompilerParams(use_tc_tiling_on_sc=False)


def _deg_body(dst_hbm, ew_hbm, out_hbm, idx_v, val_v, zb, acc):
    c = lax.axis_index("c")
    s = lax.axis_index("s")
    wid = c * NS + s
    for i in range(NROW // 16):
        zb[pl.ds(i * 16, 16)] = jnp.zeros((16,), jnp.float32)
    pltpu.sync_copy(zb, acc.at[pl.ds(s * NROW, NROW)])
    plsc.subcore_barrier()
    pltpu.sync_copy(dst_hbm.at[wid], idx_v)
    pltpu.sync_copy(ew_hbm.at[wid], val_v)

    def body(j, carry):
        pltpu.sync_copy(val_v.at[j], acc.at[idx_v.at[j]], add=True)
        return carry

    lax.fori_loop(0, NCH32, body, 0)
    plsc.subcore_barrier()
    sl = pl.ds(s * NROW, NROW)
    pltpu.sync_copy(acc.at[sl], out_hbm.at[c, sl])


_deg = pl.kernel(
    _deg_body,
    out_type=jax.ShapeDtypeStruct((NC, N_PAD), jnp.float32),
    mesh=_MESH,
    scratch_types=[
        pltpu.VMEM((NCH32, C), jnp.int32),
        pltpu.VMEM((NCH32, C), jnp.float32),
        pltpu.VMEM((NROW,), jnp.float32),
        pltpu.VMEM_SHARED((N_PAD,), jnp.float32),
    ],
    compiler_params=_SC_PARAMS,
)


def _agg_body(g_hbm, src_hbm, dst_hbm, ew_hbm, out_hbm,
              src_v, dst_v, ew_v, rows_v, zrow, acc, sem):
    c = lax.axis_index("c")
    s = lax.axis_index("s")
    for r in range(64):
        for dd in range(DH // 16):
            zrow[r, pl.ds(dd * 16, 16)] = jnp.zeros((16,), jnp.float32)
    base = s * NROW
    for k in range(NROW // 64):
        pltpu.sync_copy(zrow, acc.at[pl.ds(base + k * 64, 64)])
    plsc.subcore_barrier()

    pltpu.sync_copy(src_hbm.at[s], src_v)
    pltpu.sync_copy(dst_hbm.at[s], dst_v)
    pltpu.sync_copy(ew_hbm.at[s], ew_v)
    off = c * N

    def body(j, carry):
        for gi in range(C // 16):
            sl = pl.ds(gi * 16, 16)
            src_v[j, sl] = src_v[j, sl] + off
        pltpu.async_copy(g_hbm.at[src_v.at[j]], rows_v, sem).wait()
        for gi in range(C // 16):
            wv = ew_v[j, pl.ds(gi * 16, 16)]
            for l in range(16):
                w = wv[l]
                r = gi * 16 + l
                for dd in range(DH // 16):
                    sl = pl.ds(dd * 16, 16)
                    rows_v[r, sl] = rows_v[r, sl] * w
        pltpu.sync_copy(rows_v, acc.at[dst_v.at[j]], add=True)
        return carry

    lax.fori_loop(0, NCH16, body, 0)
    plsc.subcore_barrier()
    sl = pl.ds(base, NROW)
    pltpu.sync_copy(acc.at[sl], out_hbm.at[c, sl])


_agg = pl.kernel(
    _agg_body,
    out_type=jax.ShapeDtypeStruct((NC, N_PAD, DH), jnp.float32),
    mesh=_MESH,
    scratch_types=[
        pltpu.VMEM((NCH16, C), jnp.int32),
        pltpu.VMEM((NCH16, C), jnp.int32),
        pltpu.VMEM((NCH16, C), jnp.float32),
        pltpu.VMEM((C, DH), jnp.float32),
        pltpu.VMEM((64, DH), jnp.float32),
        pltpu.VMEM_SHARED((N_PAD, DH), jnp.float32),
        pltpu.SemaphoreType.DMA,
    ],
    compiler_params=_SC_PARAMS,
)


def _tc_a_body(degt_ref, x_ref, w1_ref, dinv_ref, h1_ref, g_ref):
    d = degt_ref[:, 0:1] + degt_ref[:, 1:2] + 1.0
    dinv = lax.rsqrt(d)
    dinv_ref[...] = dinv
    h = jnp.dot(x_ref[...], w1_ref[...], preferred_element_type=jnp.float32)
    h1_ref[...] = h
    g = h * dinv
    g_ref[0] = g[:, :DH]
    g_ref[1] = g[:, DH:]


def _tc_a(degt, x, W1):
    return pl.pallas_call(
        _tc_a_body,
        grid=(N // RB,),
        in_specs=[
            pl.BlockSpec((RB, 2), lambda i: (i, 0)),
            pl.BlockSpec((RB, D), lambda i: (i, 0)),
            pl.BlockSpec((D, D), lambda i: (0, 0)),
        ],
        out_specs=[
            pl.BlockSpec((RB, 1), lambda i: (i, 0)),
            pl.BlockSpec((RB, D), lambda i: (i, 0)),
            pl.BlockSpec((NC, RB, DH), lambda i: (0, i, 0)),
        ],
        out_shape=[
            jax.ShapeDtypeStruct((N, 1), jnp.float32),
            jax.ShapeDtypeStruct((N, D), jnp.float32),
            jax.ShapeDtypeStruct((NC, N, DH), jnp.float32),
        ],
    )(degt, x, W1)


def _tc_b_body(p_ref, h1_ref, dinv_ref, b1_ref, w2_ref, h2_ref, g_ref):
    dinv = dinv_ref[...]
    pa = jnp.concatenate([p_ref[0], p_ref[1]], axis=1)
    z = dinv * pa + (dinv * dinv) * h1_ref[...] + b1_ref[...]
    z = jnp.maximum(z, 0.0)
    h2 = jnp.dot(z, w2_ref[...], preferred_element_type=jnp.float32)
    h2_ref[...] = h2
    g = h2 * dinv
    g_ref[0] = g[:, :DH]
    g_ref[1] = g[:, DH:]


def _tc_b(p, h1, dinv, b1, W2):
    return pl.pallas_call(
        _tc_b_body,
        grid=(N // RB,),
        in_specs=[
            pl.BlockSpec((NC, RB, DH), lambda i: (0, i, 0)),
            pl.BlockSpec((RB, D), lambda i: (i, 0)),
            pl.BlockSpec((RB, 1), lambda i: (i, 0)),
            pl.BlockSpec((1, D), lambda i: (0, 0)),
            pl.BlockSpec((D, D), lambda i: (0, 0)),
        ],
        out_specs=[
            pl.BlockSpec((RB, D), lambda i: (i, 0)),
            pl.BlockSpec((NC, RB, DH), lambda i: (0, i, 0)),
        ],
        out_shape=[
            jax.ShapeDtypeStruct((N, D), jnp.float32),
            jax.ShapeDtypeStruct((NC, N, DH), jnp.float32),
        ],
    )(p, h1, dinv, b1, W2)


def _tc_c_body(q_ref, h2_ref, dinv_ref, b2_ref, w3_ref, b3_ref, out_ref):
    dinv = dinv_ref[...]
    qa = jnp.concatenate([q_ref[0], q_ref[1]], axis=1)
    z = dinv * qa + (dinv * dinv) * h2_ref[...] + b2_ref[...]
    out_ref[...] = jnp.dot(z, w3_ref[...], preferred_element_type=jnp.float32) + b3_ref[...]


def _tc_c(q, h2, dinv, b2, W3, b3):
    return pl.pallas_call(
        _tc_c_body,
        grid=(N // RB,),
        in_specs=[
            pl.BlockSpec((NC, RB, DH), lambda i: (0, i, 0)),
            pl.BlockSpec((RB, D), lambda i: (i, 0)),
            pl.BlockSpec((RB, 1), lambda i: (i, 0)),
            pl.BlockSpec((1, D), lambda i: (0, 0)),
            pl.BlockSpec((D, D), lambda i: (0, 0)),
            pl.BlockSpec((1, D), lambda i: (0, 0)),
        ],
        out_specs=pl.BlockSpec((RB, D), lambda i: (i, 0)),
        out_shape=jax.ShapeDtypeStruct((N, D), jnp.float32),
    )(q, h2, dinv, b2, W3, b3)


def kernel(x, edge_index, edge_weights, W1, b1, W2, b2, W3, b3):
    src16 = edge_index[0].reshape(NS, NCH16, C)
    dst16 = edge_index[1].reshape(NS, NCH16, C)
    ew16 = edge_weights.reshape(NS, NCH16, C)
    dst32 = edge_index[1].reshape(NW, NCH32, C)
    ew32 = edge_weights.reshape(NW, NCH32, C)

    degp = _deg(dst32, ew32)
    degt = degp[:, :N].T
    dinv, h1, g1 = _tc_a(degt, x, W1)
    p = _agg(g1.reshape(NC * N, DH), src16, dst16, ew16)
    h2, g2 = _tc_b(p, h1, dinv, b1.reshape(1, D), W2)
    q = _agg(g2.reshape(NC * N, DH), src16, dst16, ew16)
    return _tc_c(q, h2, dinv, b2.reshape(1, D), W3, b3.reshape(1, D))
